# baseline (speedup 1.0000x reference)
import jax
import jax.numpy as jnp
from jax import lax
from jax.experimental import pallas as pl
from jax.experimental.pallas import tpu as pltpu
from jax.experimental.pallas import tpu_sc as plsc

_VOCAB = 100000
_MAX_LEN = 200
_D = 64
_BATCH = 4096

_NC = 2
_NS = 16
_NW = _NC * _NS

_PER_W = _BATCH // _NW
_NBUF = 2
_N_ROUNDS = _PER_W // _NBUF


def _body(x_ref, tok_ref, pos_ref, out_ref, idx_v, rows_v, pos_v, *sems):
    in_sems = sems[:_NBUF]
    out_sems = sems[_NBUF:]
    wid = lax.axis_index("c") * _NS + lax.axis_index("s")
    b0 = wid * _PER_W

    pltpu.sync_copy(x_ref.at[pl.ds(b0, _PER_W)], idx_v)
    pltpu.sync_copy(pos_ref, pos_v)

    def gathers_start(i, s):
        for j in range(2):
            pltpu.async_copy(
                tok_ref.at[idx_v.at[i, j]],
                rows_v.at[s, pl.ds(j * 100, 100)], in_sems[s])

    def gathers_wait(i, s):
        for j in range(2):
            pltpu.make_async_copy(
                tok_ref.at[idx_v.at[i, j]],
                rows_v.at[s, pl.ds(j * 100, 100)], in_sems[s]).wait()

    def store_start(b, s):
        pltpu.async_copy(rows_v.at[s], out_ref.at[b], out_sems[s])

    def store_wait(s):
        pltpu.make_async_copy(rows_v.at[s], out_ref.at[0], out_sems[s]).wait()

    gathers_start(0, 0)

    def round_body(r, _):
        for s in range(_NBUF):
            i = r * _NBUF + s

            @pl.when(i >= 1)
            def _():
                store_wait(1 - s)

            @pl.when(i + 1 < _PER_W)
            def _():
                gathers_start(i + 1, 1 - s)

            gathers_wait(i, s)

            @plsc.parallel_loop(0, _MAX_LEN, step=1, unroll=4)
            def row_loop(row):
                for k in range(_D // 16):
                    sl = pl.ds(k * 16, 16)
                    rows_v[s, row, sl] += pos_v[row, sl]

            store_start(b0 + i, s)
        return 0

    lax.fori_loop(0, _N_ROUNDS, round_body, 0)

    store_wait((_PER_W - 1) % _NBUF)


@jax.jit
def kernel(x, token_table, pos_table):
    x3 = x.astype(jnp.int32).reshape(_BATCH, 2, _MAX_LEN // 2)
    mesh = plsc.VectorSubcoreMesh(core_axis_name="c", subcore_axis_name="s")
    out = pl.kernel(
        _body,
        out_type=jax.ShapeDtypeStruct((_BATCH, _MAX_LEN, _D), jnp.float32),
        mesh=mesh,
        scratch_types=[
            pltpu.VMEM((_PER_W, 2, _MAX_LEN // 2), jnp.int32),
            pltpu.VMEM((_NBUF, _MAX_LEN, _D), jnp.float32),
            pltpu.VMEM((_MAX_LEN, _D), jnp.float32),
        ] + [pltpu.SemaphoreType.DMA] * (2 * _NBUF),
        compiler_params=pltpu.CompilerParams(use_tc_tiling_on_sc=False),
    )(x3, token_table, pos_table)
    return out


# speedup vs baseline: 1.0591x; 1.0591x over previous
import jax
import jax.numpy as jnp
from jax import lax
from jax.experimental import pallas as pl
from jax.experimental.pallas import tpu as pltpu
from jax.experimental.pallas import tpu_sc as plsc

_VOCAB = 100000
_MAX_LEN = 200
_D = 64
_BATCH = 4096

_NC = 2
_NS = 16
_NW = _NC * _NS

_N_ROWS = _BATCH * _MAX_LEN
_PER_W = _N_ROWS // _NW
_CHUNK = 100
_N_CHUNKS = _PER_W // _CHUNK
_NBUF = 4
_LOOK = 2
_N_ROUNDS = _N_CHUNKS // _NBUF


def _body(x_ref, tok_ref, pos_ref, out_ref, idx_v, rows_v, pos_v, *sems):
    in_sems = sems[:_NBUF]
    out_sems = sems[_NBUF:]
    wid = lax.axis_index("c") * _NS + lax.axis_index("s")
    cbase = wid * _N_CHUNKS

    pltpu.sync_copy(x_ref.at[wid], idx_v)
    pltpu.sync_copy(pos_ref, pos_v)

    def gather_start(g, b):
        pltpu.async_copy(tok_ref.at[idx_v.at[g]], rows_v.at[b], in_sems[b])

    def gather_wait(g, b):
        pltpu.make_async_copy(
            tok_ref.at[idx_v.at[g]], rows_v.at[b], in_sems[b]).wait()

    def store_start(g, b):
        pltpu.async_copy(rows_v.at[b], out_ref.at[cbase + g], out_sems[b])

    def store_wait(b):
        pltpu.make_async_copy(rows_v.at[b], out_ref.at[0], out_sems[b]).wait()

    gather_start(0, 0)
    gather_start(1, 1)

    def round_body(r, _):
        for b in range(_NBUF):
            g = r * _NBUF + b
            bi = (b + _LOOK) % _NBUF
            @pl.when(g >= _LOOK)
            def _():
                store_wait(bi)

            @pl.when(g + _LOOK < _N_CHUNKS)
            def _():
                gather_start(g + _LOOK, bi)
            gather_wait(g, b)

            po = (b % 2) * _CHUNK

            @plsc.parallel_loop(0, _CHUNK, step=1, unroll=8)
            def add_row(row):
                for k in range(_D // 16):
                    sl = pl.ds(k * 16, 16)
                    rows_v[b, row, sl] += pos_v[po + row, sl]

            store_start(g, b)
        return 0

    lax.fori_loop(0, _N_ROUNDS, round_body, 0)

    store_wait((_N_CHUNKS - 2) % _NBUF)
    store_wait((_N_CHUNKS - 1) % _NBUF)


@jax.jit
def kernel(x, token_table, pos_table):
    x3 = x.astype(jnp.int32).reshape(_NW, _N_CHUNKS, _CHUNK)
    mesh = plsc.VectorSubcoreMesh(core_axis_name="c", subcore_axis_name="s")
    out = pl.kernel(
        _body,
        out_type=jax.ShapeDtypeStruct((_NW * _N_CHUNKS, _CHUNK, _D), jnp.float32),
        mesh=mesh,
        scratch_types=[
            pltpu.VMEM((_N_CHUNKS, _CHUNK), jnp.int32),
            pltpu.VMEM((_NBUF, _CHUNK, _D), jnp.float32),
            pltpu.VMEM((_MAX_LEN, _D), jnp.float32),
        ] + [pltpu.SemaphoreType.DMA] * (2 * _NBUF),
        compiler_params=pltpu.CompilerParams(use_tc_tiling_on_sc=False),
    )(x3, token_table, pos_table)
    return out.reshape(_BATCH, _MAX_LEN, _D)
